# baseline (speedup 1.0000x reference)
import jax
import jax.numpy as jnp
import numpy as np
from jax import lax
from jax.experimental import pallas as pl
from jax.experimental.pallas import tpu as pltpu
from jax.experimental.pallas import tpu_sc as plsc

R, C = 128, 32768
K = 64
NC, NS, LANES = 2, 16, 16
NW = NC * NS
ROWS_PER_W = R // NW
NV = C // LANES
HALF = C // 2
REG_B = HALF + LANES

_SIGN = np.uint32(0x80000000)
_ALL1 = np.uint32(0xFFFFFFFF)


def _process_row(xb, cand):
    G = 8

    def p1(i, accs):
        base = i * (LANES * G)
        return tuple(
            jnp.maximum(a, xb[pl.ds(base + g * LANES, LANES)])
            for g, a in enumerate(accs)
        )

    ninf = jnp.full((LANES,), -jnp.inf, jnp.float32)
    accs = lax.fori_loop(0, NV // G, p1, (ninf,) * G)
    m01 = jnp.minimum(jnp.minimum(accs[0], accs[1]),
                      jnp.minimum(accs[2], accs[3]))
    m23 = jnp.minimum(jnp.minimum(accs[4], accs[5]),
                      jnp.minimum(accs[6], accs[7]))
    lb = -jnp.max(-jnp.minimum(m01, m23))

    def p2(i, offs):
        oa, ob = offs
        ba = i * (LANES * G2)
        bb = HALF + ba
        va = [xb[pl.ds(ba + g * LANES, LANES)] for g in range(G2)]
        vb = [xb[pl.ds(bb + g * LANES, LANES)] for g in range(G2)]
        ma = [v >= lb for v in va]
        mb = [v >= lb for v in vb]
        pa = [plsc.all_reduce_population_count(m)[0] for m in ma]
        pb = [plsc.all_reduce_population_count(m)[0] for m in mb]
        for g in range(G2):
            plsc.store_compressed(cand.at[pl.ds(oa, LANES)], va[g], mask=ma[g])
            plsc.store_compressed(cand.at[pl.ds(ob, LANES)], vb[g], mask=mb[g])
            oa = oa + pa[g]
            ob = ob + pb[g]
        return oa, ob

    G2 = 4
    cnt_a, off_b = lax.fori_loop(0, HALF // (LANES * G2), p2,
                                 (jnp.int32(0), jnp.int32(REG_B)))
    cnt_b = off_b - REG_B

    nvb = (cnt_b + LANES - 1) // LANES

    def compact(j, c2):
        v = cand[pl.ds(REG_B + j * LANES, LANES)]
        cand[pl.ds(cnt_a + j * LANES, LANES)] = v
        return c2

    lax.fori_loop(0, nvb, compact, 0)
    cnt = cnt_a + cnt_b
    nv = (cnt + LANES - 1) // LANES

    lane = lax.iota(jnp.int32, LANES)

    def conv(j, c2):
        v = cand[pl.ds(j * LANES, LANES)]
        su = plsc.bitcast(v, jnp.uint32)
        uk = su ^ jnp.where(v >= 0.0, _SIGN, _ALL1)
        uk = jnp.where(lane < (cnt - j * LANES), uk, jnp.uint32(0))
        cand[pl.ds(j * LANES, LANES)] = plsc.bitcast(uk, jnp.float32)
        return c2

    lax.fori_loop(0, nv, conv, 0)

    def bit_step(b, t):
        tc = t | lax.shift_left(np.uint32(1),
                                np.uint32(31) - b.astype(jnp.uint32))

        def cstep(j, acc):
            u = plsc.bitcast(cand[pl.ds(j * LANES, LANES)], jnp.uint32)
            return acc + (u >= tc).astype(jnp.int32)

        acc = lax.fori_loop(0, nv, cstep, jnp.zeros((LANES,), jnp.int32))
        return jnp.where(jnp.sum(acc) >= K, tc, t)

    t = lax.fori_loop(0, 32, bit_step, jnp.uint32(0))

    tv = jnp.full((LANES,), t, jnp.uint32)
    sv = jnp.where(tv < _SIGN, ~tv, tv ^ _SIGN)
    tf = plsc.bitcast(sv, jnp.float32)

    def p4(i, c2):
        base = i * (LANES * G)
        vs = [xb[pl.ds(base + g * LANES, LANES)] for g in range(G)]
        os_ = [jnp.where(v >= tf, jnp.maximum(v, 0.0), 0.0) for v in vs]
        for g in range(G):
            xb[pl.ds(base + g * LANES, LANES)] = os_[g]
        return c2

    lax.fori_loop(0, NV // G, p4, 0)


def _tec_body(x_hbm, out_hbm, xb0, xb1, cand, si0, si1, so0, so1):
    wid = lax.axis_index("s") * NC + lax.axis_index("c")
    r0 = wid * ROWS_PER_W
    bufs = (xb0, xb1)
    isems = (si0, si1)
    osems = (so0, so1)

    pltpu.async_copy(x_hbm.at[r0], xb0, si0)
    for k in range(ROWS_PER_W):
        xb = bufs[k % 2]
        pltpu.make_async_copy(x_hbm.at[r0 + k], xb, isems[k % 2]).wait()
        if k + 1 < ROWS_PER_W:
            nxt = (k + 1) % 2
            if k >= 1:
                pltpu.make_async_copy(bufs[nxt], out_hbm.at[r0 + k - 1],
                                      osems[nxt]).wait()
            pltpu.async_copy(x_hbm.at[r0 + k + 1], bufs[nxt], isems[nxt])
        _process_row(xb, cand)
        pltpu.async_copy(xb, out_hbm.at[r0 + k], osems[k % 2])

    pltpu.make_async_copy(xb0, out_hbm.at[r0 + 2], so0).wait()
    pltpu.make_async_copy(xb1, out_hbm.at[r0 + 3], so1).wait()


@jax.jit
def kernel(x):
    f = pl.kernel(
        _tec_body,
        out_type=jax.ShapeDtypeStruct((R, C), jnp.float32),
        mesh=plsc.VectorSubcoreMesh(core_axis_name="c", subcore_axis_name="s"),
        compiler_params=pltpu.CompilerParams(needs_layout_passes=False),
        scratch_types=[
            pltpu.VMEM((C,), jnp.float32),
            pltpu.VMEM((C,), jnp.float32),
            pltpu.VMEM((C + 2 * LANES,), jnp.float32),
            pltpu.SemaphoreType.DMA,
            pltpu.SemaphoreType.DMA,
            pltpu.SemaphoreType.DMA,
            pltpu.SemaphoreType.DMA,
        ],
    )
    return f(x)
